# baseline (speedup 1.0000x reference)
import functools

import jax
import jax.numpy as jnp
from jax import lax
from jax.experimental import pallas as pl
from jax.experimental.pallas import tpu as pltpu
from jax.experimental.pallas import tpu_sc as plsc

NC = 2
NS = 16
LANES = 16
NW = NC * NS
K = 128


def _sc_mesh():
    return plsc.VectorSubcoreMesh(
        core_axis_name="c", subcore_axis_name="s", num_cores=NC, num_subcores=NS
    )


def _make_deg_fn(E_pad, N_pad):
    PT = E_pad // NW
    n_chunks = PT // K

    def body(col_hbm, out_hbm, hist, cidx, isem):
        c = lax.axis_index("c")
        s = lax.axis_index("s")
        tid = c * NS + s

        pltpu.async_copy(col_hbm.at[pl.ds(tid * PT, PT)], cidx, isem)

        def zfill(i, carry):
            hist[pl.ds(i * LANES, LANES)] = jnp.zeros((LANES,), jnp.float32)
            return carry

        lax.fori_loop(0, N_pad // LANES, zfill, 0)
        pltpu.make_async_copy(col_hbm.at[pl.ds(0, PT)], cidx, isem).wait()

        ones_v = jnp.ones((LANES,), jnp.float32)

        def chunk(j, carry):
            for t in range(K // LANES):
                iv = cidx[pl.ds(j * K + t * LANES, LANES)]
                plsc.addupdate_scatter(hist, [iv], ones_v)
            return carry

        lax.fori_loop(0, n_chunks, chunk, 0)
        pltpu.sync_copy(hist, out_hbm.at[pl.ds(tid * N_pad, N_pad)])

    return pl.kernel(
        body,
        out_type=jax.ShapeDtypeStruct((NW * N_pad,), jnp.float32),
        mesh=_sc_mesh(),
        compiler_params=pltpu.CompilerParams(needs_layout_passes=False),
        scratch_types=[
            pltpu.VMEM((N_pad,), jnp.float32),
            pltpu.VMEM((PT,), jnp.int32),
            pltpu.SemaphoreType.DMA,
        ],
    )


NBUF = 4
KM = 32


def _make_scatter_fn(E_pad, N_tab, N_acc, C):
    PT = E_pad // NS
    n_chunks = PT // KM
    H = N_tab // 2
    RT = H // NS
    RA = N_acc // NS

    def body(g_hbm, row_hbm, col_hbm, out_hbm, *refs):
        rows = refs[0:2]
        ridx = refs[2:2 + NBUF]
        cidx = refs[2 + NBUF:2 + 2 * NBUF]
        tab = refs[2 + 2 * NBUF]
        acc = refs[3 + 2 * NBUF]
        gsem = refs[4 + 2 * NBUF:6 + 2 * NBUF]
        ssem = refs[6 + 2 * NBUF:8 + 2 * NBUF]
        isem = refs[8 + 2 * NBUF:8 + 3 * NBUF]
        c = lax.axis_index("c")
        s = lax.axis_index("s")
        lo = c * H

        pltpu.sync_copy(
            g_hbm.at[pl.ds(lo + s * RT, RT)], tab.at[pl.ds(s * RT, RT)]
        )

        def zfill(i, carry):
            for t in range(C // LANES):
                rows[0][i, pl.ds(t * LANES, LANES)] = jnp.zeros((LANES,), jnp.float32)
            return carry

        lax.fori_loop(0, KM, zfill, 0)
        abase = s * RA
        for k in range(RA // KM):
            pltpu.sync_copy(rows[0], acc.at[pl.ds(abase + k * KM, KM)])
        rem = RA % KM
        if rem:
            pltpu.sync_copy(
                rows[0].at[pl.ds(0, rem)], acc.at[pl.ds(abase + (RA // KM) * KM, rem)]
            )
        plsc.subcore_barrier()

        ebase = c * (PT * NS) + s * PT

        def wait_sem(sem, p):
            pltpu.make_async_copy(g_hbm.at[pl.ds(0, KM)], rows[p], sem).wait()

        def wait_idx(x):
            pltpu.make_async_copy(row_hbm.at[pl.ds(0, KM)], ridx[x], isem[x]).wait()
            pltpu.make_async_copy(col_hbm.at[pl.ds(0, KM)], cidx[x], isem[x]).wait()

        for q in range(2):
            pltpu.sync_copy(row_hbm.at[pl.ds(ebase + q * KM, KM)], ridx[q])
            pltpu.sync_copy(col_hbm.at[pl.ds(ebase + q * KM, KM)], cidx[q])
        pltpu.async_copy(tab.at[ridx[0]], rows[0], gsem[0])

        def super_chunk(jo, carry):
            j = jo * NBUF
            for qq in range(NBUF):
                jj = j + qq
                p = qq % 2
                x1 = (qq + 1) % NBUF
                x2 = (qq + 2) % NBUF

                wait_sem(gsem[p], p)
                pltpu.async_copy(rows[p], acc.at[cidx[qq]], ssem[p], add=True)

                @pl.when(jj + 1 < n_chunks)
                def _():
                    @pl.when(jj >= 1)
                    def _():
                        wait_sem(ssem[1 - p], 1 - p)
                        wait_idx(x1)

                    pltpu.async_copy(tab.at[ridx[x1]], rows[1 - p], gsem[1 - p])

                @pl.when(jj + 2 < n_chunks)
                def _():
                    base = ebase + (jj + 2) * KM
                    pltpu.async_copy(row_hbm.at[pl.ds(base, KM)], ridx[x2], isem[x2])
                    pltpu.async_copy(col_hbm.at[pl.ds(base, KM)], cidx[x2], isem[x2])

            return carry

        lax.fori_loop(0, n_chunks // NBUF, super_chunk, 0)

        for p in range(2):
            wait_sem(ssem[p], p)
        plsc.subcore_barrier()
        pltpu.sync_copy(
            acc.at[pl.ds(abase, RA)],
            out_hbm.at[pl.ds(c * N_acc + abase, RA)],
        )

    return pl.kernel(
        body,
        out_type=jax.ShapeDtypeStruct((NC * N_acc, C), jnp.float32),
        mesh=_sc_mesh(),
        scratch_types=(
            [pltpu.VMEM((KM, C), jnp.float32) for _ in range(2)]
            + [pltpu.VMEM((KM,), jnp.int32) for _ in range(2 * NBUF)]
            + [
                pltpu.VMEM_SHARED((H, C), jnp.float32),
                pltpu.VMEM_SHARED((N_acc, C), jnp.float32),
            ]
            + [pltpu.SemaphoreType.DMA for _ in range(4 + NBUF)]
        ),
    )


def _dinv_from_hists(hblk):
    deg = jnp.sum(hblk, axis=0)
    return jnp.where(deg > 0, lax.rsqrt(jnp.maximum(deg, 1.0)), 0.0)


def _matmul(x, W, BLK=1024):
    N, C = x.shape

    def body(x_ref, w_ref, h_ref):
        h_ref[...] = lax.dot_general(
            x_ref[...], w_ref[...], (((1,), (1,)), ((), ())),
            preferred_element_type=jnp.float32,
        )

    return pl.pallas_call(
        body,
        grid=(N // BLK,),
        in_specs=[
            pl.BlockSpec((BLK, C), lambda i: (i, 0)),
            pl.BlockSpec((C, C), lambda i: (0, 0)),
        ],
        out_specs=pl.BlockSpec((BLK, C), lambda i: (i, 0)),
        out_shape=jax.ShapeDtypeStruct((N, C), jnp.float32),
    )(x, W)


def _edge_transform(row2, col2, H, N_acc, EB=4096):
    nblk = row2.shape[0]

    def body(r_ref, c_ref, rt_ref, ct_ref):
        rv = r_ref[0]
        cv = c_ref[0]
        dump = (N_acc - 16) + lax.broadcasted_iota(jnp.int32, (1, EB), 1) % 16
        for c in range(NC):
            rs = rv - c * H
            act = (rs >= 0) & (rs < H)
            rt_ref[c, 0] = jnp.where(act, rs, 0)
            ct_ref[c, 0] = jnp.where(act, cv, dump)

    return pl.pallas_call(
        body,
        grid=(nblk,),
        in_specs=[
            pl.BlockSpec((1, 1, EB), lambda i: (i, 0, 0)),
            pl.BlockSpec((1, 1, EB), lambda i: (i, 0, 0)),
        ],
        out_specs=[
            pl.BlockSpec((NC, 1, 1, EB), lambda i: (0, i, 0, 0)),
            pl.BlockSpec((NC, 1, 1, EB), lambda i: (0, i, 0, 0)),
        ],
        out_shape=[
            jax.ShapeDtypeStruct((NC, nblk, 1, EB), jnp.int32),
            jax.ShapeDtypeStruct((NC, nblk, 1, EB), jnp.int32),
        ],
    )(row2, col2)


def _scale(h, hists, BLK=1024):
    N, C = h.shape

    def body(h_ref, d_ref, g_ref):
        dinv = _dinv_from_hists(d_ref[...])
        g_ref[...] = h_ref[...] * dinv[:, None]

    return pl.pallas_call(
        body,
        grid=(N // BLK,),
        in_specs=[
            pl.BlockSpec((BLK, C), lambda i: (i, 0)),
            pl.BlockSpec((NW, BLK), lambda i: (0, i)),
        ],
        out_specs=pl.BlockSpec((BLK, C), lambda i: (i, 0)),
        out_shape=jax.ShapeDtypeStruct((N, C), jnp.float32),
    )(h, hists)


def _pool_loss(parts, hists, batch3, y2, b2, B, BLK=1024):
    _, _, C = parts.shape
    nblk = batch3.shape[0]

    def body(p0, p1, h_ref, bt, y_ref, b_ref, pooled_ref, loss_ref, pacc, cacc):
        i = pl.program_id(0)

        @pl.when(i == 0)
        def _():
            pacc[...] = jnp.zeros_like(pacc)
            cacc[...] = jnp.zeros_like(cacc)

        dinv = _dinv_from_hists(h_ref[...])
        outb = jnp.maximum(
            (p0[0] + p1[0]) * dinv[:, None] + b_ref[...], 0.0
        )
        bt_v = bt[0, 0, :]
        onehot = (
            bt_v[None, :] == lax.broadcasted_iota(jnp.int32, (B, BLK), 0)
        ).astype(jnp.float32)
        pacc[...] += lax.dot_general(
            onehot, outb, (((1,), (0,)), ((), ())),
            preferred_element_type=jnp.float32,
        )
        cacc[...] += jnp.sum(onehot, axis=1, keepdims=True)

        @pl.when(i == nblk - 1)
        def _():
            pooled = pacc[...] / jnp.maximum(cacc[...], 1.0)
            pooled_ref[...] = pooled
            m = jnp.max(pooled, axis=1, keepdims=True)
            lse = m + jnp.log(jnp.sum(jnp.exp(pooled - m), axis=1, keepdims=True))
            logp = pooled - lse
            oy = (
                lax.broadcasted_iota(jnp.int32, (B, C), 1) == y_ref[0][:, None]
            ).astype(jnp.float32)
            nll = -jnp.sum(logp * oy, axis=1, keepdims=True)
            loss_ref[...] = jnp.mean(nll).reshape(1, 1)

    return pl.pallas_call(
        body,
        grid=(nblk,),
        in_specs=[
            pl.BlockSpec((1, BLK, C), lambda i: (0, i, 0)),
            pl.BlockSpec((1, BLK, C), lambda i: (1, i, 0)),
            pl.BlockSpec((NW, BLK), lambda i: (0, i)),
            pl.BlockSpec((1, 1, BLK), lambda i: (i, 0, 0)),
            pl.BlockSpec((1, B), lambda i: (0, 0)),
            pl.BlockSpec((1, C), lambda i: (0, 0)),
        ],
        out_specs=[
            pl.BlockSpec((B, C), lambda i: (0, 0)),
            pl.BlockSpec((1, 1), lambda i: (0, 0)),
        ],
        out_shape=[
            jax.ShapeDtypeStruct((B, C), jnp.float32),
            jax.ShapeDtypeStruct((1, 1), jnp.float32),
        ],
        scratch_shapes=[
            pltpu.VMEM((B, C), jnp.float32),
            pltpu.VMEM((B, 1), jnp.float32),
        ],
    )(parts, parts, hists, batch3, y2, b2)


def kernel(x, edge_index, y, batch, W, b):
    N, C = x.shape
    E = edge_index.shape[1]
    B = y.shape[0]

    row = edge_index[0].astype(jnp.int32)
    col = edge_index[1].astype(jnp.int32)

    E_pad = -(-E // (NW * K)) * (NW * K)
    N_tab = -(-(N + 1) // (NS * K)) * (NS * K)
    N_acc = 10112

    pad = E_pad - E
    row_p = jnp.concatenate([row, jnp.zeros((pad,), jnp.int32)])
    col_p = jnp.concatenate([col, jnp.full((pad,), N, jnp.int32)])

    x_p = jnp.concatenate([x, jnp.zeros((N_tab - N, C), x.dtype)])
    h = _matmul(x_p, W)
    hists = _make_deg_fn(E_pad, N_tab)(col_p).reshape(NW, N_tab)
    g = _scale(h, hists)
    EB = 4096
    rt, ct = _edge_transform(
        row_p.reshape(E_pad // EB, 1, EB), col_p.reshape(E_pad // EB, 1, EB),
        N_tab // 2, N_acc, EB,
    )
    parts = _make_scatter_fn(E_pad, N_tab, N_acc, C)(
        g, rt.reshape(NC * E_pad), ct.reshape(NC * E_pad)
    )
    parts = parts.reshape(NC, N_acc, C)

    BLK = N_acc
    batch_p = jnp.concatenate(
        [batch.astype(jnp.int32), jnp.full((N_acc - N,), B, jnp.int32)]
    )
    batch3 = batch_p.reshape(N_acc // BLK, 1, BLK)
    y2 = y.astype(jnp.int32).reshape(1, B)
    pooled, loss11 = _pool_loss(parts, hists, batch3, y2, b.reshape(1, C), B, BLK)
    return pooled, loss11[0, 0]


# speedup vs baseline: 1.0991x; 1.0991x over previous
import functools

import jax
import jax.numpy as jnp
from jax import lax
from jax.experimental import pallas as pl
from jax.experimental.pallas import tpu as pltpu
from jax.experimental.pallas import tpu_sc as plsc

NC = 2
NS = 16
LANES = 16
NW = NC * NS
K = 128


def _sc_mesh():
    return plsc.VectorSubcoreMesh(
        core_axis_name="c", subcore_axis_name="s", num_cores=NC, num_subcores=NS
    )


def _make_deg_fn(E_pad, N_pad):
    PT = E_pad // NW
    n_chunks = PT // K

    def body(col_hbm, out_hbm, hist, cidx, isem):
        c = lax.axis_index("c")
        s = lax.axis_index("s")
        tid = c * NS + s

        pltpu.async_copy(col_hbm.at[pl.ds(tid * PT, PT)], cidx, isem)

        def zfill(i, carry):
            hist[pl.ds(i * LANES, LANES)] = jnp.zeros((LANES,), jnp.float32)
            return carry

        lax.fori_loop(0, N_pad // LANES, zfill, 0)
        pltpu.make_async_copy(col_hbm.at[pl.ds(0, PT)], cidx, isem).wait()

        ones_v = jnp.ones((LANES,), jnp.float32)

        def chunk(j, carry):
            for t in range(K // LANES):
                iv = cidx[pl.ds(j * K + t * LANES, LANES)]
                plsc.addupdate_scatter(hist, [iv], ones_v)
            return carry

        lax.fori_loop(0, n_chunks, chunk, 0)
        pltpu.sync_copy(hist, out_hbm.at[pl.ds(tid * N_pad, N_pad)])

    return pl.kernel(
        body,
        out_type=jax.ShapeDtypeStruct((NW * N_pad,), jnp.float32),
        mesh=_sc_mesh(),
        compiler_params=pltpu.CompilerParams(needs_layout_passes=False),
        scratch_types=[
            pltpu.VMEM((N_pad,), jnp.float32),
            pltpu.VMEM((PT,), jnp.int32),
            pltpu.SemaphoreType.DMA,
        ],
    )


NBUF = 4
KM = 32


def _make_scatter_fn(E_pad, N_tab, N_acc, C):
    PT = E_pad // NS
    n_chunks = PT // KM
    H = N_tab // 2
    RT = H // NS
    RA = N_acc // NS

    def body(g_hbm, row_hbm, col_hbm, out_hbm, *refs):
        rows = refs[0:2]
        ridx = refs[2:2 + NBUF]
        cidx = refs[2 + NBUF:2 + 2 * NBUF]
        tab = refs[2 + 2 * NBUF]
        acc = refs[3 + 2 * NBUF]
        gsem = refs[4 + 2 * NBUF:6 + 2 * NBUF]
        ssem = refs[6 + 2 * NBUF:8 + 2 * NBUF]
        isem = refs[8 + 2 * NBUF:8 + 3 * NBUF]
        c = lax.axis_index("c")
        s = lax.axis_index("s")
        lo = c * H

        pltpu.sync_copy(
            g_hbm.at[pl.ds(lo + s * RT, RT)], tab.at[pl.ds(s * RT, RT)]
        )

        def zfill(i, carry):
            for t in range(C // LANES):
                rows[0][i, pl.ds(t * LANES, LANES)] = jnp.zeros((LANES,), jnp.float32)
            return carry

        lax.fori_loop(0, KM, zfill, 0)
        abase = s * RA
        for k in range(RA // KM):
            pltpu.sync_copy(rows[0], acc.at[pl.ds(abase + k * KM, KM)])
        rem = RA % KM
        if rem:
            pltpu.sync_copy(
                rows[0].at[pl.ds(0, rem)], acc.at[pl.ds(abase + (RA // KM) * KM, rem)]
            )
        plsc.subcore_barrier()

        ebase = s * PT
        dump = jnp.full((LANES,), N_acc - 16, jnp.int32) + lax.iota(jnp.int32, 16)

        def wait_sem(sem, p):
            pltpu.make_async_copy(g_hbm.at[pl.ds(0, KM)], rows[p], sem).wait()

        def wait_idx(x):
            pltpu.make_async_copy(row_hbm.at[pl.ds(0, KM)], ridx[x], isem[x]).wait()
            pltpu.make_async_copy(col_hbm.at[pl.ds(0, KM)], cidx[x], isem[x]).wait()

        def transform(x):
            for t in range(KM // LANES):
                sl = pl.ds(t * LANES, LANES)
                rv = ridx[x][sl] - lo
                cv = cidx[x][sl]
                act = (rv >= 0) & (rv < H)
                ridx[x][sl] = jnp.where(act, rv, 0)
                cidx[x][sl] = jnp.where(act, cv, dump)

        for q in range(2):
            pltpu.sync_copy(row_hbm.at[pl.ds(ebase + q * KM, KM)], ridx[q])
            pltpu.sync_copy(col_hbm.at[pl.ds(ebase + q * KM, KM)], cidx[q])
        transform(0)
        pltpu.async_copy(tab.at[ridx[0]], rows[0], gsem[0])

        def super_chunk(jo, carry):
            j = jo * NBUF
            for qq in range(NBUF):
                jj = j + qq
                p = qq % 2
                x1 = (qq + 1) % NBUF
                x2 = (qq + 2) % NBUF

                wait_sem(gsem[p], p)
                pltpu.async_copy(rows[p], acc.at[cidx[qq]], ssem[p], add=True)

                @pl.when(jj + 1 < n_chunks)
                def _():
                    @pl.when(jj >= 1)
                    def _():
                        wait_sem(ssem[1 - p], 1 - p)
                        wait_idx(x1)

                    transform(x1)
                    pltpu.async_copy(tab.at[ridx[x1]], rows[1 - p], gsem[1 - p])

                @pl.when(jj + 2 < n_chunks)
                def _():
                    base = ebase + (jj + 2) * KM
                    pltpu.async_copy(row_hbm.at[pl.ds(base, KM)], ridx[x2], isem[x2])
                    pltpu.async_copy(col_hbm.at[pl.ds(base, KM)], cidx[x2], isem[x2])

            return carry

        lax.fori_loop(0, n_chunks // NBUF, super_chunk, 0)

        for p in range(2):
            wait_sem(ssem[p], p)
        plsc.subcore_barrier()
        pltpu.sync_copy(
            acc.at[pl.ds(abase, RA)],
            out_hbm.at[pl.ds(c * N_acc + abase, RA)],
        )

    return pl.kernel(
        body,
        out_type=jax.ShapeDtypeStruct((NC * N_acc, C), jnp.float32),
        mesh=_sc_mesh(),
        scratch_types=(
            [pltpu.VMEM((KM, C), jnp.float32) for _ in range(2)]
            + [pltpu.VMEM((KM,), jnp.int32) for _ in range(2 * NBUF)]
            + [
                pltpu.VMEM_SHARED((H, C), jnp.float32),
                pltpu.VMEM_SHARED((N_acc, C), jnp.float32),
            ]
            + [pltpu.SemaphoreType.DMA for _ in range(4 + NBUF)]
        ),
    )


def _dinv_from_hists(hblk):
    deg = jnp.sum(hblk, axis=0)
    return jnp.where(deg > 0, lax.rsqrt(jnp.maximum(deg, 1.0)), 0.0)


def _matmul(x, W, BLK=1024):
    N, C = x.shape

    def body(x_ref, w_ref, h_ref):
        h_ref[...] = lax.dot_general(
            x_ref[...], w_ref[...], (((1,), (1,)), ((), ())),
            preferred_element_type=jnp.float32,
        )

    return pl.pallas_call(
        body,
        grid=(N // BLK,),
        in_specs=[
            pl.BlockSpec((BLK, C), lambda i: (i, 0)),
            pl.BlockSpec((C, C), lambda i: (0, 0)),
        ],
        out_specs=pl.BlockSpec((BLK, C), lambda i: (i, 0)),
        out_shape=jax.ShapeDtypeStruct((N, C), jnp.float32),
    )(x, W)


def _scale(h, hists, BLK=1024):
    N, C = h.shape

    def body(h_ref, d_ref, g_ref):
        dinv = _dinv_from_hists(d_ref[...])
        g_ref[...] = h_ref[...] * dinv[:, None]

    return pl.pallas_call(
        body,
        grid=(N // BLK,),
        in_specs=[
            pl.BlockSpec((BLK, C), lambda i: (i, 0)),
            pl.BlockSpec((NW, BLK), lambda i: (0, i)),
        ],
        out_specs=pl.BlockSpec((BLK, C), lambda i: (i, 0)),
        out_shape=jax.ShapeDtypeStruct((N, C), jnp.float32),
    )(h, hists)


def _pool_loss(parts, hists, batch3, y2, b2, B, BLK=1024):
    _, _, C = parts.shape
    nblk = batch3.shape[0]

    def body(p0, p1, h_ref, bt, y_ref, b_ref, pooled_ref, loss_ref, pacc, cacc):
        i = pl.program_id(0)

        @pl.when(i == 0)
        def _():
            pacc[...] = jnp.zeros_like(pacc)
            cacc[...] = jnp.zeros_like(cacc)

        dinv = _dinv_from_hists(h_ref[...])
        outb = jnp.maximum(
            (p0[0] + p1[0]) * dinv[:, None] + b_ref[...], 0.0
        )
        bt_v = bt[0, 0, :]
        onehot = (
            bt_v[None, :] == lax.broadcasted_iota(jnp.int32, (B, BLK), 0)
        ).astype(jnp.float32)
        pacc[...] += lax.dot_general(
            onehot, outb, (((1,), (0,)), ((), ())),
            preferred_element_type=jnp.float32,
        )
        cacc[...] += jnp.sum(onehot, axis=1, keepdims=True)

        @pl.when(i == nblk - 1)
        def _():
            pooled = pacc[...] / jnp.maximum(cacc[...], 1.0)
            pooled_ref[...] = pooled
            m = jnp.max(pooled, axis=1, keepdims=True)
            lse = m + jnp.log(jnp.sum(jnp.exp(pooled - m), axis=1, keepdims=True))
            logp = pooled - lse
            oy = (
                lax.broadcasted_iota(jnp.int32, (B, C), 1) == y_ref[0][:, None]
            ).astype(jnp.float32)
            nll = -jnp.sum(logp * oy, axis=1, keepdims=True)
            loss_ref[...] = jnp.mean(nll).reshape(1, 1)

    return pl.pallas_call(
        body,
        grid=(nblk,),
        in_specs=[
            pl.BlockSpec((1, BLK, C), lambda i: (0, i, 0)),
            pl.BlockSpec((1, BLK, C), lambda i: (1, i, 0)),
            pl.BlockSpec((NW, BLK), lambda i: (0, i)),
            pl.BlockSpec((1, 1, BLK), lambda i: (i, 0, 0)),
            pl.BlockSpec((1, B), lambda i: (0, 0)),
            pl.BlockSpec((1, C), lambda i: (0, 0)),
        ],
        out_specs=[
            pl.BlockSpec((B, C), lambda i: (0, 0)),
            pl.BlockSpec((1, 1), lambda i: (0, 0)),
        ],
        out_shape=[
            jax.ShapeDtypeStruct((B, C), jnp.float32),
            jax.ShapeDtypeStruct((1, 1), jnp.float32),
        ],
        scratch_shapes=[
            pltpu.VMEM((B, C), jnp.float32),
            pltpu.VMEM((B, 1), jnp.float32),
        ],
    )(parts, parts, hists, batch3, y2, b2)


def kernel(x, edge_index, y, batch, W, b):
    N, C = x.shape
    E = edge_index.shape[1]
    B = y.shape[0]

    row = edge_index[0].astype(jnp.int32)
    col = edge_index[1].astype(jnp.int32)

    E_pad = -(-E // (NW * K)) * (NW * K)
    N_tab = -(-(N + 1) // (NS * K)) * (NS * K)
    N_acc = 10112

    pad = E_pad - E
    row_p = jnp.concatenate([row, jnp.zeros((pad,), jnp.int32)])
    col_p = jnp.concatenate([col, jnp.full((pad,), N, jnp.int32)])

    x_p = jnp.concatenate([x, jnp.zeros((N_tab - N, C), x.dtype)])
    h = _matmul(x_p, W)
    hists = _make_deg_fn(E_pad, N_tab)(col_p).reshape(NW, N_tab)
    g = _scale(h, hists)
    parts = _make_scatter_fn(E_pad, N_tab, N_acc, C)(g, row_p, col_p)
    parts = parts.reshape(NC, N_acc, C)

    BLK = N_acc
    batch_p = jnp.concatenate(
        [batch.astype(jnp.int32), jnp.full((N_acc - N,), B, jnp.int32)]
    )
    batch3 = batch_p.reshape(N_acc // BLK, 1, BLK)
    y2 = y.astype(jnp.int32).reshape(1, B)
    pooled, loss11 = _pool_loss(parts, hists, batch3, y2, b.reshape(1, C), B, BLK)
    return pooled, loss11[0, 0]
